# baseline (speedup 1.0000x reference)
import numpy as np
import jax
import jax.numpy as jnp
from jax.experimental import pallas as pl

_MAX_WAVELENGTH = 10000.0


def _sine_pe_np(seq_len: int, dim: int) -> np.ndarray:
    position = np.arange(seq_len, dtype=np.float64)
    min_freq = 1.0 / _MAX_WAVELENGTH
    timescales = np.power(
        min_freq,
        (2 * (np.arange(dim) // 2)).astype(np.float64) / float(dim),
    )
    angles = position[:, None] * timescales[None, :]
    cos_mask = (np.arange(dim) % 2).astype(np.float64)
    pe = np.sin(angles) * (1.0 - cos_mask) + np.cos(angles) * cos_mask
    return pe.astype(np.float32)


def _add_body(x_ref, pe_ref, o_ref):
    o_ref[...] = x_ref[...] + pe_ref[...]


def kernel(inputs, table):
    batch, seq_len, dim = inputs.shape
    pe = jnp.asarray(_sine_pe_np(seq_len, dim)[None])

    bb = 64
    grid = (batch // bb,)
    out = pl.pallas_call(
        _add_body,
        grid=grid,
        in_specs=[
            pl.BlockSpec((bb, seq_len, dim), lambda i: (i, 0, 0)),
            pl.BlockSpec((1, seq_len, dim), lambda i: (0, 0, 0)),
        ],
        out_specs=pl.BlockSpec((bb, seq_len, dim), lambda i: (i, 0, 0)),
        out_shape=jax.ShapeDtypeStruct((batch, seq_len, dim), jnp.float32),
    )(inputs, pe)
    return out


# speedup vs baseline: 1.0633x; 1.0633x over previous
import numpy as np
import jax
from jax import lax
import jax.numpy as jnp
from jax.experimental import pallas as pl
from jax.experimental.pallas import tpu as pltpu

_MAX_WAVELENGTH = 10000.0


def _sine_pe_np(seq_len: int, dim: int) -> np.ndarray:
    position = np.arange(seq_len, dtype=np.float64)
    min_freq = 1.0 / _MAX_WAVELENGTH
    timescales = np.power(
        min_freq,
        (2 * (np.arange(dim) // 2)).astype(np.float64) / float(dim),
    )
    angles = position[:, None] * timescales[None, :]
    cos_mask = (np.arange(dim) % 2).astype(np.float64)
    pe = np.sin(angles) * (1.0 - cos_mask) + np.cos(angles) * cos_mask
    return pe.astype(np.float32)


_NBUF = 3


def _make_body(bb, nbuf):
    def body(x_hbm, pe_ref, o_hbm, bin_ref, bout_ref, sin, sout):
        i = pl.program_id(0)
        n = pl.num_programs(0)

        def in_copy(j, slot):
            return pltpu.make_async_copy(
                x_hbm.at[pl.ds(j * bb, bb)], bin_ref.at[slot], sin.at[slot])

        def out_copy(j, slot):
            return pltpu.make_async_copy(
                bout_ref.at[slot], o_hbm.at[pl.ds(j * bb, bb)], sout.at[slot])

        slot = lax.rem(i, nbuf)

        @pl.when(i == 0)
        def _():
            for s in range(nbuf - 1):
                in_copy(s, s).start()

        nxt = i + nbuf - 1

        @pl.when(nxt < n)
        def _():
            in_copy(nxt, lax.rem(nxt, nbuf)).start()

        in_copy(i, slot).wait()

        @pl.when(i >= nbuf)
        def _():
            out_copy(i - nbuf, slot).wait()

        bout_ref[slot] = bin_ref[slot] + pe_ref[...]
        out_copy(i, slot).start()

        @pl.when(i == n - 1)
        def _():
            for k in range(nbuf):
                j = n - nbuf + k
                out_copy(j, lax.rem(j, nbuf)).wait()

    return body


def kernel(inputs, table):
    batch, seq_len, dim = inputs.shape
    pe = jnp.asarray(_sine_pe_np(seq_len, dim)[None])

    bb = 128
    nbuf = _NBUF
    grid = (batch // bb,)
    out = pl.pallas_call(
        _make_body(bb, nbuf),
        grid=grid,
        in_specs=[
            pl.BlockSpec(memory_space=pl.ANY),
            pl.BlockSpec((1, seq_len, dim), lambda i: (0, 0, 0)),
        ],
        out_specs=pl.BlockSpec(memory_space=pl.ANY),
        out_shape=jax.ShapeDtypeStruct((batch, seq_len, dim), jnp.float32),
        scratch_shapes=[
            pltpu.VMEM((nbuf, bb, seq_len, dim), jnp.float32),
            pltpu.VMEM((nbuf, bb, seq_len, dim), jnp.float32),
            pltpu.SemaphoreType.DMA((nbuf,)),
            pltpu.SemaphoreType.DMA((nbuf,)),
        ],
    )(inputs, pe)
    return out
